# baseline (speedup 1.0000x reference)
import functools

import jax
import jax.numpy as jnp
from jax import lax
from jax.experimental import pallas as pl
from jax.experimental.pallas import tpu as pltpu
from jax.experimental.pallas import tpu_sc as plsc

VOCAB = 100000
EMB = 128
EPS = 1e-12
B = 1024
S = 200
N = B * S

NUM_CORES = 2
NUM_SUBCORES = 16
NW = NUM_CORES * NUM_SUBCORES
TOK_PER_W = N // NW
CHUNK = 128
K_CHUNKS = TOK_PER_W // CHUNK
NF = EMB // 16
INV_EMB = 1.0 / EMB
import numpy as _np
_GDN = lax.GatherDimensionNumbers(
    offset_dims=(), collapsed_slice_dims=(0,), start_index_map=(0,))


def _lane_shuffle(x, perm):
    return lax.gather(x, perm, _GDN, slice_sizes=(1,),
                      mode=lax.GatherScatterMode.PROMISE_IN_BOUNDS)


def _sc_embed_ln(ids_flat, idx2_flat, word_table, bias_tbl, consts):
    mesh = plsc.VectorSubcoreMesh(core_axis_name="c", subcore_axis_name="s")

    @functools.partial(
        pl.kernel,
        mesh=mesh,
        out_type=jax.ShapeDtypeStruct((N, EMB), jnp.float32),
        scratch_types=[
            pltpu.VMEM((TOK_PER_W,), jnp.int32),
            pltpu.VMEM((TOK_PER_W,), jnp.int32),
            pltpu.VMEM((2, EMB), jnp.float32),
            pltpu.VMEM((CHUNK, EMB), jnp.float32),
            pltpu.VMEM((CHUNK, EMB), jnp.float32),
            pltpu.VMEM((CHUNK, EMB), jnp.float32),
            pltpu.VMEM((CHUNK, EMB), jnp.float32),
            pltpu.VMEM((CHUNK, EMB), jnp.float32),
            pltpu.VMEM((CHUNK, EMB), jnp.float32),
            pltpu.SemaphoreType.DMA,
            pltpu.SemaphoreType.DMA,
            pltpu.SemaphoreType.DMA,
            pltpu.SemaphoreType.DMA,
            pltpu.SemaphoreType.DMA,
            pltpu.SemaphoreType.DMA,
            pltpu.SemaphoreType.DMA,
            pltpu.SemaphoreType.DMA,
            pltpu.SemaphoreType.DMA,
        ],
    )
    def body(ids_hbm, idx2_hbm, table_hbm, bias_hbm, consts_hbm, out_hbm,
             ids_v, idx2_v, consts_v,
             rows0, rows1, rows2, bias0, bias1, bias2,
             gsem0, gsem1, gsem2, bsem0, bsem1, bsem2, ssem0, ssem1, ssem2):
        wid = lax.axis_index("s") * NUM_CORES + lax.axis_index("c")
        w_base = wid * TOK_PER_W
        rows = (rows0, rows1, rows2)
        bias = (bias0, bias1, bias2)
        gsem = (gsem0, gsem1, gsem2)
        bsem = (bsem0, bsem1, bsem2)
        ssem = (ssem0, ssem1, ssem2)

        def gather_copy(c, b):
            idx = ids_v.at[pl.ds(c * CHUNK, CHUNK)]
            return pltpu.make_async_copy(table_hbm.at[idx], rows[b], gsem[b])

        def bias_copy(c, b):
            idx = idx2_v.at[pl.ds(c * CHUNK, CHUNK)]
            return pltpu.make_async_copy(bias_hbm.at[idx], bias[b], bsem[b])

        def store_copy(c, b):
            return pltpu.make_async_copy(
                rows[b], out_hbm.at[pl.ds(w_base + c * CHUNK, CHUNK)], ssem[b])

        def start_chunk(c, b):
            gather_copy(c, b).start()
            bias_copy(c, b).start()

        def wait_chunk(c, b):
            gather_copy(c, b).wait()
            bias_copy(c, b).wait()

        pltpu.sync_copy(ids_hbm.at[pl.ds(w_base, TOK_PER_W)], ids_v)
        pltpu.sync_copy(idx2_hbm.at[pl.ds(w_base, TOK_PER_W)], idx2_v)
        pltpu.sync_copy(consts_hbm, consts_v)

        gv = [consts_v[0, pl.ds(f * 16, 16)] for f in range(NF)]
        bv = [consts_v[1, pl.ds(f * 16, 16)] for f in range(NF)]

        def compute_chunk(c, rowbuf, biasbuf):
            @plsc.parallel_loop(0, CHUNK, unroll=8)
            def tok_body(i):
                xs = []
                for f in range(NF):
                    sl = pl.ds(f * 16, 16)
                    xs.append(rowbuf[i, sl] + biasbuf[i, sl])
                s1 = xs[0]
                s2 = xs[0] * xs[0]
                for f in range(1, NF):
                    s1 = s1 + xs[f]
                    s2 = s2 + xs[f] * xs[f]
                iota = lax.iota(jnp.int32, 16)
                for k in (8, 4, 2, 1):
                    p = lax.reshape(iota ^ k, (16, 1))
                    s1 = s1 + _lane_shuffle(s1, p)
                    s2 = s2 + _lane_shuffle(s2, p)
                mean = s1 * INV_EMB
                m2 = s2 * INV_EMB
                var = m2 - mean * mean + EPS
                half = var * 0.5
                yi = lax.bitcast_convert_type(var, jnp.int32)
                yi = 0x5F3759DF - lax.shift_right_logical(yi, 1)
                y = lax.bitcast_convert_type(yi, jnp.float32)
                for _ in range(3):
                    y = y * (1.5 - half * y * y)
                for f in range(NF):
                    sl = pl.ds(f * 16, 16)
                    rowbuf[i, sl] = (xs[f] - mean) * y * gv[f] + bv[f]

        def chunk_body(c, b):
            wait_chunk(c, b)
            compute_chunk(c, rows[b], bias[b])
            store_copy(c, b).start()
            bp = (b + 2) % 3
            store_copy(c - 1, bp).wait()

            @pl.when(c + 2 < K_CHUNKS)
            def _():
                start_chunk(c + 2, bp)

        start_chunk(0, 0)
        start_chunk(1, 1)
        wait_chunk(0, 0)
        compute_chunk(0, rows0, bias0)
        store_copy(0, 0).start()
        start_chunk(2, 2)
        wait_chunk(1, 1)
        compute_chunk(1, rows1, bias1)
        store_copy(1, 1).start()
        store_copy(0, 0).wait()
        start_chunk(3, 0)

        def triple(k3, carry):
            c0 = 3 * k3 + 2
            chunk_body(c0, 2)
            chunk_body(c0 + 1, 0)
            chunk_body(c0 + 2, 1)
            return carry

        lax.fori_loop(0, (K_CHUNKS - 2) // 3, triple, 0)

        store_copy(K_CHUNKS - 1, 1).wait()

    return body(ids_flat, idx2_flat, word_table, bias_tbl, consts)


def kernel(input_ids, token_type_ids, word_table, pos_table, type_table, gamma, beta):
    ids_flat = input_ids.reshape(-1)
    idx2 = (2 * jnp.arange(S, dtype=jnp.int32)[None, :]
            + token_type_ids).reshape(-1)
    bias_tbl = (pos_table[:S, None, :] + type_table[None, :, :]).reshape(2 * S, EMB)
    consts = jnp.stack([gamma, beta])
    out = _sc_embed_ln(ids_flat, idx2, word_table, bias_tbl, consts)
    return out.reshape(B, S, EMB)


# speedup vs baseline: 1.4730x; 1.4730x over previous
import functools

import jax
import jax.numpy as jnp
from jax import lax
from jax.experimental import pallas as pl
from jax.experimental.pallas import tpu as pltpu
from jax.experimental.pallas import tpu_sc as plsc

VOCAB = 100000
EMB = 128
EPS = 1e-12
B = 1024
S = 200
N = B * S

NUM_CORES = 2
NUM_SUBCORES = 16
NW = NUM_CORES * NUM_SUBCORES
TOK_PER_W = N // NW
CHUNK = 128
K_CHUNKS = TOK_PER_W // CHUNK
NF = EMB // 16
INV_EMB = 1.0 / EMB
import numpy as _np
_GDN = lax.GatherDimensionNumbers(
    offset_dims=(), collapsed_slice_dims=(0,), start_index_map=(0,))


def _lane_shuffle(x, perm):
    return lax.gather(x, perm, _GDN, slice_sizes=(1,),
                      mode=lax.GatherScatterMode.PROMISE_IN_BOUNDS)


def _sc_embed_ln(ids_flat, idx2_flat, word_table, bias_tbl, consts):
    mesh = plsc.VectorSubcoreMesh(core_axis_name="c", subcore_axis_name="s")

    @functools.partial(
        pl.kernel,
        mesh=mesh,
        out_type=jax.ShapeDtypeStruct((N, EMB), jnp.float32),
        scratch_types=[
            pltpu.VMEM((TOK_PER_W,), jnp.int32),
            pltpu.VMEM((TOK_PER_W,), jnp.int32),
            pltpu.VMEM((2, EMB), jnp.float32),
            pltpu.VMEM((CHUNK, EMB), jnp.float32),
            pltpu.VMEM((CHUNK, EMB), jnp.float32),
            pltpu.VMEM((CHUNK, EMB), jnp.float32),
            pltpu.VMEM((CHUNK, EMB), jnp.float32),
            pltpu.VMEM((CHUNK, EMB), jnp.float32),
            pltpu.VMEM((CHUNK, EMB), jnp.float32),
            pltpu.SemaphoreType.DMA,
            pltpu.SemaphoreType.DMA,
            pltpu.SemaphoreType.DMA,
            pltpu.SemaphoreType.DMA,
            pltpu.SemaphoreType.DMA,
            pltpu.SemaphoreType.DMA,
            pltpu.SemaphoreType.DMA,
            pltpu.SemaphoreType.DMA,
            pltpu.SemaphoreType.DMA,
        ],
    )
    def body(ids_hbm, idx2_hbm, table_hbm, bias_hbm, consts_hbm, out_hbm,
             ids_v, idx2_v, consts_v,
             rows0, rows1, rows2, bias0, bias1, bias2,
             gsem0, gsem1, gsem2, bsem0, bsem1, bsem2, ssem0, ssem1, ssem2):
        wid = lax.axis_index("s") * NUM_CORES + lax.axis_index("c")
        w_base = wid * TOK_PER_W
        rows = (rows0, rows1, rows2)
        bias = (bias0, bias1, bias2)
        gsem = (gsem0, gsem1, gsem2)
        bsem = (bsem0, bsem1, bsem2)
        ssem = (ssem0, ssem1, ssem2)

        def gather_copy(c, b):
            idx = ids_v.at[pl.ds(c * CHUNK, CHUNK)]
            return pltpu.make_async_copy(table_hbm.at[idx], rows[b], gsem[b])

        def bias_copy(c, b):
            idx = idx2_v.at[pl.ds(c * CHUNK, CHUNK)]
            return pltpu.make_async_copy(bias_hbm.at[idx], bias[b], bsem[b])

        def store_copy(c, b):
            return pltpu.make_async_copy(
                rows[b], out_hbm.at[pl.ds(w_base + c * CHUNK, CHUNK)], ssem[b])

        def start_chunk(c, b):
            gather_copy(c, b).start()
            bias_copy(c, b).start()

        def wait_chunk(c, b):
            gather_copy(c, b).wait()
            bias_copy(c, b).wait()

        pltpu.sync_copy(ids_hbm.at[pl.ds(w_base, TOK_PER_W)], ids_v)
        pltpu.sync_copy(idx2_hbm.at[pl.ds(w_base, TOK_PER_W)], idx2_v)
        pltpu.sync_copy(consts_hbm, consts_v)

        gv = [consts_v[0, pl.ds(f * 16, 16)] for f in range(NF)]
        bv = [consts_v[1, pl.ds(f * 16, 16)] for f in range(NF)]

        def compute_chunk(c, rowbuf, biasbuf):
            @plsc.parallel_loop(0, CHUNK, unroll=4)
            def tok_body(i):
                xs = []
                for f in range(NF):
                    sl = pl.ds(f * 16, 16)
                    xs.append(rowbuf[i, sl] + biasbuf[i, sl])
                s1 = xs[0]
                s2 = xs[0] * xs[0]
                for f in range(1, NF):
                    s1 = s1 + xs[f]
                    s2 = s2 + xs[f] * xs[f]
                iota = lax.iota(jnp.int32, 16)
                for k in (8, 4, 2, 1):
                    p = lax.reshape(iota ^ k, (16, 1))
                    s1 = s1 + _lane_shuffle(s1, p)
                    s2 = s2 + _lane_shuffle(s2, p)
                mean = s1 * INV_EMB
                m2 = s2 * INV_EMB
                var = m2 - mean * mean + EPS
                half = var * 0.5
                yi = lax.bitcast_convert_type(var, jnp.int32)
                yi = 0x5F3759DF - lax.shift_right_logical(yi, 1)
                y = lax.bitcast_convert_type(yi, jnp.float32)
                for _ in range(2):
                    y = y * (1.5 - half * y * y)
                for f in range(NF):
                    sl = pl.ds(f * 16, 16)
                    rowbuf[i, sl] = (xs[f] - mean) * y * gv[f] + bv[f]

        def chunk_body(c, b):
            wait_chunk(c, b)
            compute_chunk(c, rows[b], bias[b])
            store_copy(c, b).start()
            bp = (b + 2) % 3
            store_copy(c - 1, bp).wait()

            @pl.when(c + 2 < K_CHUNKS)
            def _():
                start_chunk(c + 2, bp)

        start_chunk(0, 0)
        start_chunk(1, 1)
        wait_chunk(0, 0)
        compute_chunk(0, rows0, bias0)
        store_copy(0, 0).start()
        start_chunk(2, 2)
        wait_chunk(1, 1)
        compute_chunk(1, rows1, bias1)
        store_copy(1, 1).start()
        store_copy(0, 0).wait()
        start_chunk(3, 0)

        def triple(k3, carry):
            c0 = 3 * k3 + 2
            chunk_body(c0, 2)
            chunk_body(c0 + 1, 0)
            chunk_body(c0 + 2, 1)
            return carry

        lax.fori_loop(0, (K_CHUNKS - 2) // 3, triple, 0)

        store_copy(K_CHUNKS - 1, 1).wait()

    return body(ids_flat, idx2_flat, word_table, bias_tbl, consts)


def kernel(input_ids, token_type_ids, word_table, pos_table, type_table, gamma, beta):
    ids_flat = input_ids.reshape(-1)
    idx2 = (2 * jnp.arange(S, dtype=jnp.int32)[None, :]
            + token_type_ids).reshape(-1)
    bias_tbl = (pos_table[:S, None, :] + type_table[None, :, :]).reshape(2 * S, EMB)
    consts = jnp.stack([gamma, beta])
    out = _sc_embed_ln(ids_flat, idx2, word_table, bias_tbl, consts)
    return out.reshape(B, S, EMB)
